# baseline (speedup 1.0000x reference)
import jax
import jax.numpy as jnp
from jax.experimental import pallas as pl
from jax.experimental.pallas import tpu as pltpu

N = 10000
IN_C = 128
HID_C = 32
OUT_C = 16
BR = 400


def _gcn_body(x_ref, adj_ref, w1_ref, b1_ref, w2_ref, b2_ref, out_ref,
              y_scr, z_scr):
    p = pl.program_id(0)
    r = pl.program_id(1)

    @pl.when((p == 0) & (r == 0))
    def _():
        y_scr[...] = jnp.dot(x_ref[...], w1_ref[...],
                             preferred_element_type=jnp.float32)

    @pl.when(p == 0)
    def _():
        h = jnp.dot(adj_ref[...], y_scr[...],
                    preferred_element_type=jnp.float32)
        h = jnp.maximum(h + b1_ref[...], 0.0)
        zblk = jnp.dot(h, w2_ref[...], preferred_element_type=jnp.float32)
        z_scr[pl.ds(r * BR, BR), :] = zblk
        out_ref[0] = zblk

    @pl.when(p == 1)
    def _():
        g = jnp.dot(adj_ref[...], z_scr[...],
                    preferred_element_type=jnp.float32)
        g = jnp.maximum(g + b2_ref[...], 0.0)
        m = jnp.max(g, axis=-1, keepdims=True)
        e = g - m
        lse = jnp.log(jnp.sum(jnp.exp(e), axis=-1, keepdims=True))
        out_ref[0] = e - lse


def kernel(x, adj, W1, b1, W2, b2):
    b1 = b1.reshape(1, HID_C)
    b2 = b2.reshape(1, OUT_C)
    grid = (2, N // BR)
    return pl.pallas_call(
        _gcn_body,
        grid=grid,
        in_specs=[
            pl.BlockSpec((N, IN_C), lambda p, r: (0, 0)),
            pl.BlockSpec((BR, N), lambda p, r: (r, 0)),
            pl.BlockSpec((IN_C, HID_C), lambda p, r: (0, 0)),
            pl.BlockSpec((1, HID_C), lambda p, r: (0, 0)),
            pl.BlockSpec((HID_C, OUT_C), lambda p, r: (0, 0)),
            pl.BlockSpec((1, OUT_C), lambda p, r: (0, 0)),
        ],
        out_specs=pl.BlockSpec((1, BR, OUT_C), lambda p, r: (p, r, 0)),
        out_shape=jax.ShapeDtypeStruct((2, N, OUT_C), jnp.float32),
        scratch_shapes=[
            pltpu.VMEM((N, HID_C), jnp.float32),
            pltpu.VMEM((N, OUT_C), jnp.float32),
        ],
        compiler_params=pltpu.CompilerParams(
            dimension_semantics=("arbitrary", "arbitrary"),
        ),
    )(x, adj, W1, b1, W2, b2)[1]


# speedup vs baseline: 1.0191x; 1.0191x over previous
import jax
import jax.numpy as jnp
from jax.experimental import pallas as pl
from jax.experimental.pallas import tpu as pltpu

N = 10000
IN_C = 128
HID_C = 32
OUT_C = 16
BR = 400


def _gcn_body(x_ref, adj_ref, w1_ref, b1_ref, w2_ref, b2_ref, out_ref,
              y_scr, z_scr):
    p = pl.program_id(0)
    r = pl.program_id(1)

    @pl.when((p == 0) & (r == 0))
    def _():
        y_scr[...] = jnp.dot(x_ref[...], w1_ref[...],
                             preferred_element_type=jnp.float32)

    @pl.when(p == 0)
    def _():
        h = jnp.dot(adj_ref[...].astype(jnp.bfloat16),
                    y_scr[...].astype(jnp.bfloat16),
                    preferred_element_type=jnp.float32)
        h = jnp.maximum(h + b1_ref[...], 0.0)
        zblk = jnp.dot(h, w2_ref[...], preferred_element_type=jnp.float32)
        z_scr[pl.ds(r * BR, BR), :] = zblk
        out_ref[0] = zblk

    @pl.when(p == 1)
    def _():
        g = jnp.dot(adj_ref[...].astype(jnp.bfloat16),
                    z_scr[...].astype(jnp.bfloat16),
                    preferred_element_type=jnp.float32)
        g = jnp.maximum(g + b2_ref[...], 0.0)
        m = jnp.max(g, axis=-1, keepdims=True)
        e = g - m
        lse = jnp.log(jnp.sum(jnp.exp(e), axis=-1, keepdims=True))
        out_ref[0] = e - lse


def kernel(x, adj, W1, b1, W2, b2):
    b1 = b1.reshape(1, HID_C)
    b2 = b2.reshape(1, OUT_C)
    grid = (2, N // BR)
    return pl.pallas_call(
        _gcn_body,
        grid=grid,
        in_specs=[
            pl.BlockSpec((N, IN_C), lambda p, r: (0, 0)),
            pl.BlockSpec((BR, N), lambda p, r: (r, 0)),
            pl.BlockSpec((IN_C, HID_C), lambda p, r: (0, 0)),
            pl.BlockSpec((1, HID_C), lambda p, r: (0, 0)),
            pl.BlockSpec((HID_C, OUT_C), lambda p, r: (0, 0)),
            pl.BlockSpec((1, OUT_C), lambda p, r: (0, 0)),
        ],
        out_specs=pl.BlockSpec((1, BR, OUT_C), lambda p, r: (p, r, 0)),
        out_shape=jax.ShapeDtypeStruct((2, N, OUT_C), jnp.float32),
        scratch_shapes=[
            pltpu.VMEM((N, HID_C), jnp.float32),
            pltpu.VMEM((N, OUT_C), jnp.float32),
        ],
        compiler_params=pltpu.CompilerParams(
            dimension_semantics=("arbitrary", "arbitrary"),
        ),
    )(x, adj, W1, b1, W2, b2)[1]
